# baseline (speedup 1.0000x reference)
import functools

import jax
import jax.numpy as jnp
from jax import lax
from jax.experimental import pallas as pl
from jax.experimental.pallas import tpu as pltpu
from jax.experimental.pallas import tpu_sc as plsc

INPUT_SIZE = 512
HIDDEN = 1024
EMBED_DIM = 256
NUM_EMB = 8192
BATCH = 16384

ROW_TILE = 256
GRID_ROWS = BATCH // ROW_TILE


def _encode_body(x_ref, w1_ref, b1_ref, w2_ref, b2_ref, w3_ref, b3_ref,
                 emb_ref, enorm_ref, idx_ref):
    x = x_ref[...]
    h1 = jax.nn.relu(jnp.dot(x, w1_ref[...]) + b1_ref[...])
    h2 = jax.nn.relu(jnp.dot(h1, w2_ref[...]) + b2_ref[...])
    f = jnp.dot(h2, w3_ref[...]) + b3_ref[...]
    fnorm = jnp.sum(f ** 2, axis=1, keepdims=True)
    m = lax.dot_general(f, emb_ref[...], (((1,), (1,)), ((), ())))
    d = (fnorm + enorm_ref[...]) - 2.0 * m
    dmin = jnp.min(d, axis=1, keepdims=True)
    iota = lax.broadcasted_iota(jnp.int32, d.shape, 1)
    idx = jnp.min(jnp.where(d == dmin, iota, NUM_EMB), axis=1)
    idx_ref[...] = idx.reshape(1, 1, ROW_TILE)


def _encode_indices(x, W1, b1, W2, b2, W3, b3, embeddings, enorm):
    nrows = x.shape[0]
    grid = nrows // ROW_TILE
    return pl.pallas_call(
        _encode_body,
        grid=(grid,),
        in_specs=[
            pl.BlockSpec((ROW_TILE, INPUT_SIZE), lambda i: (i, 0)),
            pl.BlockSpec((INPUT_SIZE, HIDDEN), lambda i: (0, 0)),
            pl.BlockSpec((1, HIDDEN), lambda i: (0, 0)),
            pl.BlockSpec((HIDDEN, HIDDEN), lambda i: (0, 0)),
            pl.BlockSpec((1, HIDDEN), lambda i: (0, 0)),
            pl.BlockSpec((HIDDEN, EMBED_DIM), lambda i: (0, 0)),
            pl.BlockSpec((1, EMBED_DIM), lambda i: (0, 0)),
            pl.BlockSpec((NUM_EMB, EMBED_DIM), lambda i: (0, 0)),
            pl.BlockSpec((1, NUM_EMB), lambda i: (0, 0)),
        ],
        out_specs=pl.BlockSpec((1, 1, ROW_TILE), lambda i: (i, 0, 0)),
        out_shape=jax.ShapeDtypeStruct((grid, 1, ROW_TILE), jnp.int32),
    )(x, W1, b1.reshape(1, HIDDEN), W2, b2.reshape(1, HIDDEN),
      W3, b3.reshape(1, EMBED_DIM), embeddings, enorm.reshape(1, NUM_EMB))


_SC_INFO = plsc.get_sparse_core_info()
_NC = _SC_INFO.num_cores
_NS = _SC_INFO.num_subcores
_NW = _NC * _NS
_CHUNK = 128


def _gather_rows(embeddings, idx):
    nrows = idx.shape[0]
    bpw = nrows // _NW
    nchunk = bpw // _CHUNK
    mesh = plsc.VectorSubcoreMesh(core_axis_name="c", subcore_axis_name="s")

    @functools.partial(
        pl.kernel,
        mesh=mesh,
        out_type=jax.ShapeDtypeStruct((nrows, EMBED_DIM), jnp.float32),
        scratch_types=[
            pltpu.VMEM((bpw,), jnp.int32),
            pltpu.VMEM((_CHUNK, EMBED_DIM), jnp.float32),
            pltpu.VMEM((_CHUNK, EMBED_DIM), jnp.float32),
            pltpu.SemaphoreType.DMA,
            pltpu.SemaphoreType.DMA,
        ],
    )
    def k(table_hbm, idx_hbm, out_hbm, idx_v, rows0, rows1, sem0, sem1):
        wid = lax.axis_index("s") * _NC + lax.axis_index("c")
        base = wid * bpw
        pltpu.sync_copy(idx_hbm.at[pl.ds(base, bpw)], idx_v)
        bufs = (rows0, rows1)
        sems = (sem0, sem1)
        copies = [
            pltpu.make_async_copy(
                table_hbm.at[idx_v.at[pl.ds(c * _CHUNK, _CHUNK)]],
                bufs[c % 2], sems[c % 2])
            for c in range(nchunk)
        ]
        copies[0].start()
        for c in range(nchunk):
            if c + 1 < nchunk:
                copies[c + 1].start()
            copies[c].wait()
            pltpu.sync_copy(bufs[c % 2], out_hbm.at[pl.ds(base + c * _CHUNK, _CHUNK)])

    return k(embeddings, idx)


def kernel(x, W1, b1, W2, b2, W3, b3, embeddings):
    enorm = jnp.sum(embeddings ** 2, axis=1)
    half = BATCH // 2
    args = (W1, b1, W2, b2, W3, b3, embeddings, enorm)
    idx0 = _encode_indices(x[:half], *args).reshape(half)
    q0 = _gather_rows(embeddings, idx0)
    idx1 = _encode_indices(x[half:], *args).reshape(half)
    q1 = _gather_rows(embeddings, idx1)
    return jnp.concatenate([q0, q1], axis=0)


# speedup vs baseline: 1.0249x; 1.0249x over previous
import functools

import jax
import jax.numpy as jnp
from jax import lax
from jax.experimental import pallas as pl
from jax.experimental.pallas import tpu as pltpu
from jax.experimental.pallas import tpu_sc as plsc

INPUT_SIZE = 512
HIDDEN = 1024
EMBED_DIM = 256
NUM_EMB = 8192
BATCH = 16384

ROW_TILE = 256
GRID_ROWS = BATCH // ROW_TILE


def _encode_body(x_ref, w1_ref, b1_ref, w2_ref, b2_ref, w3_ref, b3_ref,
                 emb_ref, enorm_ref, idx_ref):
    x = x_ref[...]
    h1 = jax.nn.relu(jnp.dot(x, w1_ref[...]) + b1_ref[...])
    h2 = jax.nn.relu(jnp.dot(h1, w2_ref[...]) + b2_ref[...])
    f = jnp.dot(h2, w3_ref[...]) + b3_ref[...]
    fnorm = jnp.sum(f ** 2, axis=1, keepdims=True)
    m = lax.dot_general(f, emb_ref[...], (((1,), (1,)), ((), ())))
    d = (fnorm + enorm_ref[...]) - 2.0 * m
    dmin = jnp.min(d, axis=1, keepdims=True)
    iota = lax.broadcasted_iota(jnp.int32, d.shape, 1)
    idx = jnp.min(jnp.where(d == dmin, iota, NUM_EMB), axis=1)
    idx_ref[...] = idx.reshape(1, 1, ROW_TILE)


def _encode_indices(x, W1, b1, W2, b2, W3, b3, embeddings, enorm):
    nrows = x.shape[0]
    grid = nrows // ROW_TILE
    return pl.pallas_call(
        _encode_body,
        grid=(grid,),
        in_specs=[
            pl.BlockSpec((ROW_TILE, INPUT_SIZE), lambda i: (i, 0)),
            pl.BlockSpec((INPUT_SIZE, HIDDEN), lambda i: (0, 0)),
            pl.BlockSpec((1, HIDDEN), lambda i: (0, 0)),
            pl.BlockSpec((HIDDEN, HIDDEN), lambda i: (0, 0)),
            pl.BlockSpec((1, HIDDEN), lambda i: (0, 0)),
            pl.BlockSpec((HIDDEN, EMBED_DIM), lambda i: (0, 0)),
            pl.BlockSpec((1, EMBED_DIM), lambda i: (0, 0)),
            pl.BlockSpec((NUM_EMB, EMBED_DIM), lambda i: (0, 0)),
            pl.BlockSpec((1, NUM_EMB), lambda i: (0, 0)),
        ],
        out_specs=pl.BlockSpec((1, 1, ROW_TILE), lambda i: (i, 0, 0)),
        out_shape=jax.ShapeDtypeStruct((grid, 1, ROW_TILE), jnp.int32),
    )(x, W1, b1.reshape(1, HIDDEN), W2, b2.reshape(1, HIDDEN),
      W3, b3.reshape(1, EMBED_DIM), embeddings, enorm.reshape(1, NUM_EMB))


_SC_INFO = plsc.get_sparse_core_info()
_NC = _SC_INFO.num_cores
_NS = _SC_INFO.num_subcores
_NW = _NC * _NS
_CHUNK = 128


def _gather_rows(embeddings, idx):
    nrows = idx.shape[0]
    bpw = nrows // _NW
    nchunk = bpw // _CHUNK
    mesh = plsc.VectorSubcoreMesh(core_axis_name="c", subcore_axis_name="s")

    @functools.partial(
        pl.kernel,
        mesh=mesh,
        out_type=jax.ShapeDtypeStruct((nrows, EMBED_DIM), jnp.float32),
        scratch_types=[
            pltpu.VMEM((bpw,), jnp.int32),
            pltpu.VMEM((_CHUNK, EMBED_DIM), jnp.float32),
            pltpu.VMEM((_CHUNK, EMBED_DIM), jnp.float32),
            pltpu.SemaphoreType.DMA,
            pltpu.SemaphoreType.DMA,
        ],
    )
    def k(table_hbm, idx_hbm, out_hbm, idx_v, rows0, rows1, sem0, sem1):
        wid = lax.axis_index("s") * _NC + lax.axis_index("c")
        base = wid * bpw
        pltpu.sync_copy(idx_hbm.at[pl.ds(base, bpw)], idx_v)
        bufs = (rows0, rows1)
        sems = (sem0, sem1)
        copies = [
            pltpu.make_async_copy(
                table_hbm.at[idx_v.at[pl.ds(c * _CHUNK, _CHUNK)]],
                bufs[c % 2], sems[c % 2])
            for c in range(nchunk)
        ]
        copies[0].start()
        for c in range(nchunk):
            if c + 1 < nchunk:
                copies[c + 1].start()
            copies[c].wait()
            pltpu.sync_copy(bufs[c % 2], out_hbm.at[pl.ds(base + c * _CHUNK, _CHUNK)])

    return k(embeddings, idx)


def kernel(x, W1, b1, W2, b2, W3, b3, embeddings):
    enorm = jnp.sum(embeddings ** 2, axis=1)
    idx = _encode_indices(x, W1, b1, W2, b2, W3, b3, embeddings, enorm)
    return _gather_rows(embeddings, idx.reshape(BATCH))
